# baseline (speedup 1.0000x reference)
import functools

import jax
import jax.numpy as jnp
from jax import lax
from jax.experimental import pallas as pl
from jax.experimental.pallas import tpu as pltpu
from jax.experimental.pallas import tpu_sc as plsc

NC = 2
NS = 16
EDGE_BLK = 80

_SC_PARAMS = pltpu.CompilerParams(use_tc_tiling_on_sc=False)


def _sc_mesh():
    return plsc.VectorSubcoreMesh(core_axis_name="c", subcore_axis_name="s")


ROWS_MAIN = 624


def _tile_rowcopy(s, n_nodes, copy_fn):
    tail = n_nodes - NS * ROWS_MAIN
    copy_fn(s * ROWS_MAIN, ROWS_MAIN)
    if tail > 0:
        @pl.when(s == 0)
        def _():
            copy_fn(NS * ROWS_MAIN, tail)


DEG_NBUF = 5
PROP_NBUF = 10


def _deg_pass(ei, ones_blk, zeros16, n_nodes):
    n_edges = ei.shape[1]
    ept = n_edges // (NC * NS)
    n_outer = ept // (DEG_NBUF * EDGE_BLK)

    @functools.partial(
        pl.kernel,
        out_type=jax.ShapeDtypeStruct((NC, n_nodes, 16), jnp.float32),
        mesh=_sc_mesh(),
        scratch_types=[
            pltpu.VMEM((ept,), jnp.int32),
            pltpu.VMEM((EDGE_BLK, 16), jnp.float32),
            pltpu.VMEM_SHARED((n_nodes, 16), jnp.float32),
            pltpu.SemaphoreType.DMA,
            pltpu.SemaphoreType.DMA,
        ],
        compiler_params=_SC_PARAMS,
    )
    def k(ei_hbm, ones_hbm, z_hbm, out_hbm, idx_v, ones_v, acc, isem, ssem):
        c = lax.axis_index("c")
        s = lax.axis_index("s")
        w = c * NS + s
        ld = pltpu.async_copy(ei_hbm.at[1].at[pl.ds(w * ept, ept)], idx_v, isem)
        pltpu.sync_copy(ones_hbm, ones_v)
        _tile_rowcopy(s, n_nodes, lambda r0, nr: pltpu.sync_copy(
            z_hbm.at[pl.ds(r0, nr)], acc.at[pl.ds(r0, nr)]))
        ld.wait()
        plsc.subcore_barrier()

        @pl.loop(0, n_outer)
        def _(i):
            b0 = i * DEG_NBUF * EDGE_BLK
            descs = [pltpu.async_copy(
                ones_v, acc.at[idx_v.at[pl.ds(b0 + j * EDGE_BLK, EDGE_BLK)]],
                ssem, add=True) for j in range(DEG_NBUF)]
            for d in descs:
                d.wait()

        plsc.subcore_barrier()
        _tile_rowcopy(s, n_nodes, lambda r0, nr: pltpu.sync_copy(
            acc.at[pl.ds(r0, nr)], out_hbm.at[c].at[pl.ds(r0, nr)]))

    return k(ei, ones_blk, zeros16)


PROP_BLK = 80
PROP_NBUF = 5


def _prop_pass(g_halves, ei, n_nodes, half_ch):
    n_edges = ei.shape[1]
    ept = n_edges // NS
    chunk = PROP_NBUF * PROP_BLK
    n_outer = ept // chunk

    @functools.partial(
        pl.kernel,
        out_type=jax.ShapeDtypeStruct((NC, n_nodes, half_ch), jnp.float32),
        mesh=_sc_mesh(),
        scratch_types=(
            [pltpu.VMEM((2, 2, chunk), jnp.int32)]
            + [pltpu.VMEM((PROP_BLK, half_ch), jnp.float32)
               for _ in range(PROP_NBUF)]
            + [pltpu.VMEM_SHARED((n_nodes, half_ch), jnp.float32),
               pltpu.VMEM_SHARED((n_nodes, half_ch), jnp.float32),
               pltpu.SemaphoreType.DMA,
               pltpu.SemaphoreType.DMA,
               pltpu.SemaphoreType.DMA]
        ),
        compiler_params=_SC_PARAMS,
    )
    def k(g_hbm, ei_hbm, out_hbm, eidx, *rest):
        rows = rest[:PROP_NBUF]
        acc, tbl, isem, gsem, ssem = rest[PROP_NBUF:]
        c = lax.axis_index("c")
        s = lax.axis_index("s")
        base = s * ept

        def load_idx(i, p):
            b = base + i * chunk
            return [pltpu.async_copy(ei_hbm.at[d].at[pl.ds(b, chunk)],
                                     eidx.at[d].at[p], isem)
                    for d in (0, 1)]

        first = load_idx(0, 0)
        _tile_rowcopy(s, n_nodes, lambda r0, nr: pltpu.sync_copy(
            g_hbm.at[c].at[pl.ds(r0, nr)], tbl.at[pl.ds(r0, nr)]))
        _tile_rowcopy(s, n_nodes, lambda r0, nr: pltpu.sync_copy(
            g_hbm.at[c].at[pl.ds(r0, nr)], acc.at[pl.ds(r0, nr)]))
        for d in first:
            d.wait()
        plsc.subcore_barrier()

        @pl.loop(0, n_outer)
        def _(i):
            p = lax.rem(i, 2)
            @pl.when(i + 1 < n_outer)
            def _():
                load_idx(i + 1, 1 - p)

            gds = [pltpu.async_copy(
                tbl.at[eidx.at[0].at[p].at[pl.ds(j * PROP_BLK,
                                                 PROP_BLK)]],
                rows[j], gsem) for j in range(PROP_NBUF)]
            sds = []
            for j in range(PROP_NBUF):
                gds[j].wait()
                sds.append(pltpu.async_copy(
                    rows[j],
                    acc.at[eidx.at[1].at[p].at[pl.ds(j * PROP_BLK, PROP_BLK)]],
                    ssem, add=True))
            for d in sds:
                d.wait()

            @pl.when(i + 1 < n_outer)
            def _():
                for d in (0, 1):
                    pltpu.make_async_copy(
                        ei_hbm.at[d].at[pl.ds(base, chunk)],
                        eidx.at[d].at[1 - p], isem).wait()

        plsc.subcore_barrier()
        _tile_rowcopy(s, n_nodes, lambda r0, nr: pltpu.sync_copy(
            acc.at[pl.ds(r0, nr)], out_hbm.at[c].at[pl.ds(r0, nr)]))

    return k(g_halves, ei)


def _tc_stage1(cnt, x, W1):
    n = x.shape[0]
    hc = W1.shape[1] // 2

    def body(cnt_ref, x_ref, w_ref, g_ref, dinv_ref):
        deg = cnt_ref[0, :, 0:1] + cnt_ref[1, :, 0:1] + 1.0
        dinv = lax.rsqrt(deg)
        dinv_ref[...] = dinv
        g = dinv * jnp.dot(x_ref[...], w_ref[...],
                           preferred_element_type=jnp.float32)
        g_ref[0] = g[:, :hc]
        g_ref[1] = g[:, hc:]

    return pl.pallas_call(
        body,
        out_shape=(jax.ShapeDtypeStruct((2, n, hc), jnp.float32),
                   jax.ShapeDtypeStruct((n, 1), jnp.float32)),
    )(cnt, x, W1)


def _tc_stage2(part1, dinv, b1, W_cat):
    n = dinv.shape[0]
    hc = W_cat.shape[1] // 2

    def body(p_ref, dinv_ref, b_ref, w_ref, g2_ref):
        dinv = dinv_ref[...]
        h = dinv * jnp.concatenate([p_ref[0], p_ref[1]], axis=1) + b_ref[...]
        g2 = dinv * jnp.dot(h, w_ref[...], preferred_element_type=jnp.float32)
        g2_ref[0] = g2[:, :hc]
        g2_ref[1] = g2[:, hc:]

    return pl.pallas_call(
        body,
        out_shape=jax.ShapeDtypeStruct((2, n, hc), jnp.float32),
    )(part1, dinv, b1, W_cat)


def _tc_stage3(part2, dinv, b_mu, b_ls, init_dist):
    n, oc = init_dist.shape

    def body(p_ref, dinv_ref, bmu_ref, bls_ref, init_ref, z_ref):
        dinv = dinv_ref[...]
        mu = dinv * p_ref[0] + bmu_ref[...]
        logstd = dinv * p_ref[1] + bls_ref[...]
        z_ref[...] = mu + init_ref[...] * jnp.exp(logstd)

    return pl.pallas_call(
        body,
        out_shape=jax.ShapeDtypeStruct((n, oc), jnp.float32),
    )(part2, dinv, b_mu, b_ls, init_dist)


def kernel(x, edge_index, init_dist, W1, b1, W_mu, b_mu, W_ls, b_ls):
    n, _ = x.shape
    ei = edge_index
    if ei.dtype != jnp.int32:
        ei = ei.astype(jnp.int32)
    W_cat = jnp.concatenate([W_mu, W_ls], axis=1)
    b1r = b1[None, :]
    b_mur = b_mu[None, :]
    b_lsr = b_ls[None, :]

    hc1 = W1.shape[1] // 2
    hc2 = W_cat.shape[1] // 2
    zeros16 = jnp.zeros((n, 16), jnp.float32)
    ones_blk = jnp.ones((EDGE_BLK, 16), jnp.float32)

    cnt = _deg_pass(ei, ones_blk, zeros16, n)
    g1, dinv = _tc_stage1(cnt, x, W1)
    part1 = _prop_pass(g1, ei, n, hc1)
    g2 = _tc_stage2(part1, dinv, b1r, W_cat)
    part2 = _prop_pass(g2, ei, n, hc2)
    z = _tc_stage3(part2, dinv, b_mur, b_lsr, init_dist)
    return z


# speedup vs baseline: 1.4429x; 1.4429x over previous
import functools

import jax
import jax.numpy as jnp
from jax import lax
from jax.experimental import pallas as pl
from jax.experimental.pallas import tpu as pltpu
from jax.experimental.pallas import tpu_sc as plsc

NC = 2
NS = 16
EDGE_BLK = 80

_SC_PARAMS = pltpu.CompilerParams(use_tc_tiling_on_sc=False)


def _sc_mesh():
    return plsc.VectorSubcoreMesh(core_axis_name="c", subcore_axis_name="s")


ROWS_MAIN = 624


def _tile_rowcopy(s, n_nodes, copy_fn):
    tail = n_nodes - NS * ROWS_MAIN
    copy_fn(s * ROWS_MAIN, ROWS_MAIN)
    if tail > 0:
        @pl.when(s == 0)
        def _():
            copy_fn(NS * ROWS_MAIN, tail)


DEG_NBUF = 5
PROP_NBUF = 10


def _deg_pass(ei, ones_blk, zeros16, n_nodes):
    n_edges = ei.shape[1]
    ept = n_edges // (NC * NS)
    n_outer = ept // (DEG_NBUF * EDGE_BLK)

    @functools.partial(
        pl.kernel,
        out_type=jax.ShapeDtypeStruct((NC, n_nodes, 16), jnp.float32),
        mesh=_sc_mesh(),
        scratch_types=[
            pltpu.VMEM((ept,), jnp.int32),
            pltpu.VMEM((EDGE_BLK, 16), jnp.float32),
            pltpu.VMEM_SHARED((n_nodes, 16), jnp.float32),
            pltpu.SemaphoreType.DMA,
            pltpu.SemaphoreType.DMA,
        ],
        compiler_params=_SC_PARAMS,
    )
    def k(ei_hbm, ones_hbm, z_hbm, out_hbm, idx_v, ones_v, acc, isem, ssem):
        c = lax.axis_index("c")
        s = lax.axis_index("s")
        w = c * NS + s
        ld = pltpu.async_copy(ei_hbm.at[1].at[pl.ds(w * ept, ept)], idx_v, isem)
        pltpu.sync_copy(ones_hbm, ones_v)
        _tile_rowcopy(s, n_nodes, lambda r0, nr: pltpu.sync_copy(
            z_hbm.at[pl.ds(r0, nr)], acc.at[pl.ds(r0, nr)]))
        ld.wait()
        plsc.subcore_barrier()

        @pl.loop(0, n_outer)
        def _(i):
            b0 = i * DEG_NBUF * EDGE_BLK
            descs = [pltpu.async_copy(
                ones_v, acc.at[idx_v.at[pl.ds(b0 + j * EDGE_BLK, EDGE_BLK)]],
                ssem, add=True) for j in range(DEG_NBUF)]
            for d in descs:
                d.wait()

        plsc.subcore_barrier()
        _tile_rowcopy(s, n_nodes, lambda r0, nr: pltpu.sync_copy(
            acc.at[pl.ds(r0, nr)], out_hbm.at[c].at[pl.ds(r0, nr)]))

    return k(ei, ones_blk, zeros16)


PROP_BLK = 80
PROP_NBUF = 4


def _prop_pass(g, zeros_feat, ei, n_nodes, n_ch):
    n_edges = ei.shape[1]
    ept = n_edges // (NC * NS)
    chunk = PROP_NBUF * PROP_BLK
    n_outer = ept // chunk
    tail = ept - n_outer * chunk

    @functools.partial(
        pl.kernel,
        out_type=jax.ShapeDtypeStruct((NC, n_nodes, n_ch), jnp.float32),
        mesh=_sc_mesh(),
        scratch_types=(
            [pltpu.VMEM((2, 2, chunk), jnp.int32),
             pltpu.VMEM((2, max(tail, 8)), jnp.int32)]
            + [pltpu.VMEM((PROP_BLK, n_ch), jnp.float32)
               for _ in range(PROP_NBUF)]
            + [pltpu.VMEM_SHARED((n_nodes, n_ch), jnp.float32),
               pltpu.SemaphoreType.DMA,
               pltpu.SemaphoreType.DMA,
               pltpu.SemaphoreType.DMA]
        ),
        compiler_params=_SC_PARAMS,
    )
    def k(g_hbm, z_hbm, ei_hbm, out_hbm, eidx, tidx, *rest):
        rows = rest[:PROP_NBUF]
        acc, isem, gsem, ssem = rest[PROP_NBUF:]
        c = lax.axis_index("c")
        s = lax.axis_index("s")
        base = (c * NS + s) * ept

        def load_idx(i, p):
            b = base + i * chunk
            return [pltpu.async_copy(ei_hbm.at[d].at[pl.ds(b, chunk)],
                                     eidx.at[d].at[p], isem)
                    for d in (0, 1)]

        first = load_idx(0, 0)
        @pl.when(c == 0)
        def _():
            _tile_rowcopy(s, n_nodes, lambda r0, nr: pltpu.sync_copy(
                g_hbm.at[pl.ds(r0, nr)], acc.at[pl.ds(r0, nr)]))

        @pl.when(c != 0)
        def _():
            _tile_rowcopy(s, n_nodes, lambda r0, nr: pltpu.sync_copy(
                z_hbm.at[pl.ds(r0, nr)], acc.at[pl.ds(r0, nr)]))

        for d in first:
            d.wait()
        plsc.subcore_barrier()

        @pl.loop(0, n_outer)
        def _(i):
            p = lax.rem(i, 2)
            @pl.when(i + 1 < n_outer)
            def _():
                load_idx(i + 1, 1 - p)

            gds = [pltpu.async_copy(
                g_hbm.at[eidx.at[0].at[p].at[pl.ds(j * PROP_BLK, PROP_BLK)]],
                rows[j], gsem) for j in range(PROP_NBUF)]
            sds = []
            for j in range(PROP_NBUF):
                gds[j].wait()
                sds.append(pltpu.async_copy(
                    rows[j],
                    acc.at[eidx.at[1].at[p].at[pl.ds(j * PROP_BLK, PROP_BLK)]],
                    ssem, add=True))
            for d in sds:
                d.wait()

            @pl.when(i + 1 < n_outer)
            def _():
                for d in (0, 1):
                    pltpu.make_async_copy(
                        ei_hbm.at[d].at[pl.ds(base, chunk)],
                        eidx.at[d].at[1 - p], isem).wait()

        if tail:
            bt = base + n_outer * chunk
            for d in (0, 1):
                pltpu.sync_copy(ei_hbm.at[d].at[pl.ds(bt, tail)], tidx.at[d])
            pltpu.async_copy(g_hbm.at[tidx.at[0]], rows[0].at[pl.ds(0, tail)],
                             gsem).wait()
            pltpu.sync_copy(rows[0].at[pl.ds(0, tail)], acc.at[tidx.at[1]],
                            add=True)

        plsc.subcore_barrier()
        _tile_rowcopy(s, n_nodes, lambda r0, nr: pltpu.sync_copy(
            acc.at[pl.ds(r0, nr)], out_hbm.at[c].at[pl.ds(r0, nr)]))

    return k(g, zeros_feat, ei)


def _tc_stage1(cnt, x, W1):
    n = x.shape[0]

    def body(cnt_ref, x_ref, w_ref, g_ref, dinv_ref):
        deg = cnt_ref[0, :, 0:1] + cnt_ref[1, :, 0:1] + 1.0
        dinv = lax.rsqrt(deg)
        dinv_ref[...] = dinv
        g_ref[...] = dinv * jnp.dot(x_ref[...], w_ref[...],
                                    preferred_element_type=jnp.float32)

    return pl.pallas_call(
        body,
        out_shape=(jax.ShapeDtypeStruct((n, W1.shape[1]), jnp.float32),
                   jax.ShapeDtypeStruct((n, 1), jnp.float32)),
    )(cnt, x, W1)


def _tc_stage2(part1, dinv, b1, W_cat):
    n = dinv.shape[0]

    def body(p_ref, dinv_ref, b_ref, w_ref, g2_ref):
        dinv = dinv_ref[...]
        h = dinv * (p_ref[0] + p_ref[1]) + b_ref[...]
        g2_ref[...] = dinv * jnp.dot(h, w_ref[...],
                                     preferred_element_type=jnp.float32)

    return pl.pallas_call(
        body,
        out_shape=jax.ShapeDtypeStruct((n, W_cat.shape[1]), jnp.float32),
    )(part1, dinv, b1, W_cat)


def _tc_stage3(part2, dinv, b_cat, init_dist):
    n, oc = init_dist.shape

    def body(p_ref, dinv_ref, b_ref, init_ref, z_ref):
        o = dinv_ref[...] * (p_ref[0] + p_ref[1]) + b_ref[...]
        mu = o[:, :oc]
        logstd = o[:, oc:]
        z_ref[...] = mu + init_ref[...] * jnp.exp(logstd)

    return pl.pallas_call(
        body,
        out_shape=jax.ShapeDtypeStruct((n, oc), jnp.float32),
    )(part2, dinv, b_cat, init_dist)


def kernel(x, edge_index, init_dist, W1, b1, W_mu, b_mu, W_ls, b_ls):
    n, _ = x.shape
    ei = edge_index
    if ei.dtype != jnp.int32:
        ei = ei.astype(jnp.int32)
    W_cat = jnp.concatenate([W_mu, W_ls], axis=1)
    b1r = b1[None, :]
    b_catr = jnp.concatenate([b_mu, b_ls])[None, :]

    c1 = W1.shape[1]
    c2 = W_cat.shape[1]
    zeros16 = jnp.zeros((n, 16), jnp.float32)
    ones_blk = jnp.ones((EDGE_BLK, 16), jnp.float32)
    zeros_c1 = jnp.zeros((n, c1), jnp.float32)
    zeros_c2 = jnp.zeros((n, c2), jnp.float32)

    cnt = _deg_pass(ei, ones_blk, zeros16, n)
    g1, dinv = _tc_stage1(cnt, x, W1)
    part1 = _prop_pass(g1, zeros_c1, ei, n, c1)
    g2 = _tc_stage2(part1, dinv, b1r, W_cat)
    part2 = _prop_pass(g2, zeros_c2, ei, n, c2)
    z = _tc_stage3(part2, dinv, b_catr, init_dist)
    return z


# speedup vs baseline: 1.5549x; 1.0776x over previous
import functools

import jax
import jax.numpy as jnp
from jax import lax
from jax.experimental import pallas as pl
from jax.experimental.pallas import tpu as pltpu
from jax.experimental.pallas import tpu_sc as plsc

NC = 2
NS = 16
EDGE_BLK = 80

_SC_PARAMS = pltpu.CompilerParams(use_tc_tiling_on_sc=False)


def _sc_mesh():
    return plsc.VectorSubcoreMesh(core_axis_name="c", subcore_axis_name="s")


ROWS_MAIN = 624


def _tile_rowcopy(s, n_nodes, copy_fn):
    tail = n_nodes - NS * ROWS_MAIN
    copy_fn(s * ROWS_MAIN, ROWS_MAIN)
    if tail > 0:
        @pl.when(s == 0)
        def _():
            copy_fn(NS * ROWS_MAIN, tail)


DEG_NBUF = 5
PROP_NBUF = 10


def _deg_pass(ei, ones_blk, zeros16, n_nodes):
    n_edges = ei.shape[1]
    ept = n_edges // (NC * NS)
    n_outer = ept // (DEG_NBUF * EDGE_BLK)

    @functools.partial(
        pl.kernel,
        out_type=jax.ShapeDtypeStruct((NC, n_nodes, 16), jnp.float32),
        mesh=_sc_mesh(),
        scratch_types=[
            pltpu.VMEM((ept,), jnp.int32),
            pltpu.VMEM((EDGE_BLK, 16), jnp.float32),
            pltpu.VMEM_SHARED((n_nodes, 16), jnp.float32),
            pltpu.SemaphoreType.DMA,
            pltpu.SemaphoreType.DMA,
        ],
        compiler_params=_SC_PARAMS,
    )
    def k(ei_hbm, ones_hbm, z_hbm, out_hbm, idx_v, ones_v, acc, isem, ssem):
        c = lax.axis_index("c")
        s = lax.axis_index("s")
        w = c * NS + s
        ld = pltpu.async_copy(ei_hbm.at[1].at[pl.ds(w * ept, ept)], idx_v, isem)
        pltpu.sync_copy(ones_hbm, ones_v)
        _tile_rowcopy(s, n_nodes, lambda r0, nr: pltpu.sync_copy(
            z_hbm.at[pl.ds(r0, nr)], acc.at[pl.ds(r0, nr)]))
        ld.wait()
        plsc.subcore_barrier()

        @pl.loop(0, n_outer)
        def _(i):
            b0 = i * DEG_NBUF * EDGE_BLK
            descs = [pltpu.async_copy(
                ones_v, acc.at[idx_v.at[pl.ds(b0 + j * EDGE_BLK, EDGE_BLK)]],
                ssem, add=True) for j in range(DEG_NBUF)]
            for d in descs:
                d.wait()

        plsc.subcore_barrier()
        _tile_rowcopy(s, n_nodes, lambda r0, nr: pltpu.sync_copy(
            acc.at[pl.ds(r0, nr)], out_hbm.at[c].at[pl.ds(r0, nr)]))

    return k(ei, ones_blk, zeros16)


PROP_BLK = 80
PROP_NBUF = 4


def _prop_pass(g, zeros_feat, ei, n_nodes, n_ch):
    n_edges = ei.shape[1]
    ept = n_edges // (NC * NS)
    chunk = PROP_NBUF * PROP_BLK
    n_outer = ept // chunk
    tail = ept - n_outer * chunk

    @functools.partial(
        pl.kernel,
        out_type=jax.ShapeDtypeStruct((NC, n_nodes, n_ch), jnp.float32),
        mesh=_sc_mesh(),
        scratch_types=(
            [pltpu.VMEM((2, 2, chunk), jnp.int32),
             pltpu.VMEM((2, max(tail, 8)), jnp.int32)]
            + [pltpu.VMEM((PROP_BLK, n_ch), jnp.float32)
               for _ in range(PROP_NBUF)]
            + [pltpu.VMEM_SHARED((n_nodes, n_ch), jnp.float32),
               pltpu.SemaphoreType.DMA,
               pltpu.SemaphoreType.DMA]
            + [pltpu.SemaphoreType.DMA for _ in range(PROP_NBUF)]
        ),
        compiler_params=_SC_PARAMS,
    )
    def k(g_hbm, z_hbm, ei_hbm, out_hbm, eidx, tidx, *rest):
        rows = rest[:PROP_NBUF]
        acc, isem, gsem = rest[PROP_NBUF:PROP_NBUF + 3]
        ssems = rest[PROP_NBUF + 3:]
        c = lax.axis_index("c")
        s = lax.axis_index("s")
        base = (c * NS + s) * ept

        def load_idx(i, p):
            b = base + i * chunk
            return [pltpu.async_copy(ei_hbm.at[d].at[pl.ds(b, chunk)],
                                     eidx.at[d].at[p], isem)
                    for d in (0, 1)]

        first = load_idx(0, 0)
        @pl.when(c == 0)
        def _():
            _tile_rowcopy(s, n_nodes, lambda r0, nr: pltpu.sync_copy(
                g_hbm.at[pl.ds(r0, nr)], acc.at[pl.ds(r0, nr)]))

        @pl.when(c != 0)
        def _():
            _tile_rowcopy(s, n_nodes, lambda r0, nr: pltpu.sync_copy(
                z_hbm.at[pl.ds(r0, nr)], acc.at[pl.ds(r0, nr)]))

        for d in first:
            d.wait()
        plsc.subcore_barrier()

        def consume_scatter(j, p):
            pltpu.make_async_copy(
                rows[j],
                acc.at[eidx.at[1].at[p].at[pl.ds(j * PROP_BLK, PROP_BLK)]],
                ssems[j]).wait()

        @pl.loop(0, n_outer)
        def _(i):
            p = lax.rem(i, 2)
            gds = []
            for j in range(PROP_NBUF):
                @pl.when(i > 0)
                def _():
                    consume_scatter(j, p)
                gds.append(pltpu.async_copy(
                    g_hbm.at[eidx.at[0].at[p].at[pl.ds(j * PROP_BLK,
                                                       PROP_BLK)]],
                    rows[j], gsem))

            @pl.when(i + 1 < n_outer)
            def _():
                load_idx(i + 1, 1 - p)
            for j in range(PROP_NBUF):
                gds[j].wait()
                pltpu.async_copy(
                    rows[j],
                    acc.at[eidx.at[1].at[p].at[pl.ds(j * PROP_BLK, PROP_BLK)]],
                    ssems[j], add=True)

            @pl.when(i + 1 < n_outer)
            def _():
                for d in (0, 1):
                    pltpu.make_async_copy(
                        ei_hbm.at[d].at[pl.ds(base, chunk)],
                        eidx.at[d].at[1 - p], isem).wait()

        for j in range(PROP_NBUF):
            consume_scatter(j, 0)

        if tail:
            bt = base + n_outer * chunk
            for d in (0, 1):
                pltpu.sync_copy(ei_hbm.at[d].at[pl.ds(bt, tail)], tidx.at[d])
            pltpu.async_copy(g_hbm.at[tidx.at[0]], rows[0].at[pl.ds(0, tail)],
                             gsem).wait()
            pltpu.sync_copy(rows[0].at[pl.ds(0, tail)], acc.at[tidx.at[1]],
                            add=True)

        plsc.subcore_barrier()
        _tile_rowcopy(s, n_nodes, lambda r0, nr: pltpu.sync_copy(
            acc.at[pl.ds(r0, nr)], out_hbm.at[c].at[pl.ds(r0, nr)]))

    return k(g, zeros_feat, ei)


def _tc_stage1(cnt, x, W1):
    n = x.shape[0]

    def body(cnt_ref, x_ref, w_ref, g_ref, dinv_ref):
        deg = cnt_ref[0, :, 0:1] + cnt_ref[1, :, 0:1] + 1.0
        dinv = lax.rsqrt(deg)
        dinv_ref[...] = dinv
        g_ref[...] = dinv * jnp.dot(x_ref[...], w_ref[...],
                                    preferred_element_type=jnp.float32)

    return pl.pallas_call(
        body,
        out_shape=(jax.ShapeDtypeStruct((n, W1.shape[1]), jnp.float32),
                   jax.ShapeDtypeStruct((n, 1), jnp.float32)),
    )(cnt, x, W1)


def _tc_stage2(part1, dinv, b1, W_cat):
    n = dinv.shape[0]

    def body(p_ref, dinv_ref, b_ref, w_ref, g2_ref):
        dinv = dinv_ref[...]
        h = dinv * (p_ref[0] + p_ref[1]) + b_ref[...]
        g2_ref[...] = dinv * jnp.dot(h, w_ref[...],
                                     preferred_element_type=jnp.float32)

    return pl.pallas_call(
        body,
        out_shape=jax.ShapeDtypeStruct((n, W_cat.shape[1]), jnp.float32),
    )(part1, dinv, b1, W_cat)


def _tc_stage3(part2, dinv, b_cat, init_dist):
    n, oc = init_dist.shape

    def body(p_ref, dinv_ref, b_ref, init_ref, z_ref):
        o = dinv_ref[...] * (p_ref[0] + p_ref[1]) + b_ref[...]
        mu = o[:, :oc]
        logstd = o[:, oc:]
        z_ref[...] = mu + init_ref[...] * jnp.exp(logstd)

    return pl.pallas_call(
        body,
        out_shape=jax.ShapeDtypeStruct((n, oc), jnp.float32),
    )(part2, dinv, b_cat, init_dist)


def kernel(x, edge_index, init_dist, W1, b1, W_mu, b_mu, W_ls, b_ls):
    n, _ = x.shape
    ei = edge_index
    if ei.dtype != jnp.int32:
        ei = ei.astype(jnp.int32)
    W_cat = jnp.concatenate([W_mu, W_ls], axis=1)
    b1r = b1[None, :]
    b_catr = jnp.concatenate([b_mu, b_ls])[None, :]

    c1 = W1.shape[1]
    c2 = W_cat.shape[1]
    zeros16 = jnp.zeros((n, 16), jnp.float32)
    ones_blk = jnp.ones((EDGE_BLK, 16), jnp.float32)
    zeros_c1 = jnp.zeros((n, c1), jnp.float32)
    zeros_c2 = jnp.zeros((n, c2), jnp.float32)

    cnt = _deg_pass(ei, ones_blk, zeros16, n)
    g1, dinv = _tc_stage1(cnt, x, W1)
    part1 = _prop_pass(g1, zeros_c1, ei, n, c1)
    g2 = _tc_stage2(part1, dinv, b1r, W_cat)
    part2 = _prop_pass(g2, zeros_c2, ei, n, c2)
    z = _tc_stage3(part2, dinv, b_catr, init_dist)
    return z
